# baseline (speedup 1.0000x reference)
import functools

import jax
import jax.numpy as jnp
from jax import lax
from jax.experimental import pallas as pl
from jax.experimental.pallas import tpu as pltpu
from jax.experimental.pallas import tpu_sc as plsc

EPS = 1e-7
N = 589824
NC, NS, L = 2, 16, 16
NW = NC * NS
NA = N // NW
NCHUNK = 4
CH = NA // NCHUNK
GROUPS = CH // L


def _reg_body(od_hbm, td_hbm, os_hbm, out_hbm,
              od_v0, od_v1, td_v0, td_v1, os_v0, os_v1, acc_v,
              sem0, sem1):
    wid = lax.axis_index("c") * NS + lax.axis_index("s")
    od_bufs = (od_v0, od_v1)
    td_bufs = (td_v0, td_v1)
    os_bufs = (os_v0, os_v1)
    sems = (sem0, sem1)

    def chunk_copies(g):
        slot = g % 2
        base = pl.multiple_of(wid * NA + g * CH, 8)
        base4 = pl.multiple_of(base * 4, 8)
        return (
            pltpu.make_async_copy(od_hbm.at[pl.ds(base4, 4 * CH)],
                                  od_bufs[slot], sems[slot]),
            pltpu.make_async_copy(td_hbm.at[pl.ds(base4, 4 * CH)],
                                  td_bufs[slot], sems[slot]),
            pltpu.make_async_copy(os_hbm.at[pl.ds(base, CH)],
                                  os_bufs[slot], sems[slot]),
        )

    for c in chunk_copies(0):
        c.start()

    quarter = jnp.right_shift(lax.iota(jnp.int32, L), 2)

    acc_a = jnp.zeros((L,), jnp.float32)
    acc_p = jnp.zeros((L,), jnp.float32)
    acc_m = jnp.zeros((L,), jnp.float32)

    for g in range(NCHUNK):
        slot = g % 2
        if g + 1 < NCHUNK:
            for c in chunk_copies(g + 1):
                c.start()
        for c in chunk_copies(g):
            c.wait()
        odv, tdv, osv = od_bufs[slot], td_bufs[slot], os_bufs[slot]

        def group(g2, carry, odv=odv, tdv=tdv, osv=osv):
            a, p, m = carry
            osg = osv[pl.ds(g2 * L, L)]
            p = p + jnp.where(osg > 0.0, 1.0, 0.0)
            m = m + jnp.where(osg != -1.0, 1.0, 0.0)
            for j in range(4):
                base = g2 * (4 * L) + j * L
                d = jnp.abs(odv[pl.ds(base, L)] - tdv[pl.ds(base, L)])
                mn = jnp.minimum(d, 1.0)
                l1 = 0.5 * mn * mn + (d - mn)
                osb = plsc.load_gather(osv, [g2 * L + j * 4 + quarter])
                a = a + l1 * jnp.where(osb > 0.0, 1.0, 0.0)
            return a, p, m

        acc_a, acc_p, acc_m = lax.fori_loop(
            0, GROUPS, group, (acc_a, acc_p, acc_m))

    acc_v[0, :] = acc_a
    acc_v[1, :] = acc_p
    acc_v[2, :] = acc_m
    pltpu.sync_copy(acc_v, out_hbm.at[wid])


_reg_call = pl.kernel(
    _reg_body,
    out_type=jax.ShapeDtypeStruct((NW, 3, L), jnp.float32),
    mesh=plsc.VectorSubcoreMesh(core_axis_name="c", subcore_axis_name="s"),
    compiler_params=pltpu.CompilerParams(needs_layout_passes=False),
    scratch_types=[
        pltpu.VMEM((4 * CH,), jnp.float32),
        pltpu.VMEM((4 * CH,), jnp.float32),
        pltpu.VMEM((4 * CH,), jnp.float32),
        pltpu.VMEM((4 * CH,), jnp.float32),
        pltpu.VMEM((CH,), jnp.float32),
        pltpu.VMEM((CH,), jnp.float32),
        pltpu.VMEM((3, L), jnp.float32),
        pltpu.SemaphoreType.DMA,
        pltpu.SemaphoreType.DMA,
    ],
)


ROWS = N // 128
TC_GRID = 16
TC_BLK = ROWS // TC_GRID


def _cls_body(ts_ref, os_ref, bce_ref, cnt_ref):
    i = pl.program_id(0)
    ts = ts_ref[...]
    o = jnp.clip(os_ref[...], EPS, 1.0 - EPS)
    mask = (ts != -1.0).astype(jnp.float32)
    bce = -(ts * jnp.log(o) + (1.0 - ts) * jnp.log(1.0 - o))

    @pl.when(i == 0)
    def _():
        bce_ref[0, 0] = 0.0
        cnt_ref[0, 0] = 0.0

    bce_ref[0, 0] += jnp.sum(bce * mask)
    cnt_ref[0, 0] += jnp.sum(mask)


_cls_call = pl.pallas_call(
    _cls_body,
    grid=(TC_GRID,),
    in_specs=[
        pl.BlockSpec((TC_BLK, 128), lambda i: (i, 0)),
        pl.BlockSpec((TC_BLK, 128), lambda i: (i, 0)),
    ],
    out_specs=[
        pl.BlockSpec((1, 1), lambda i: (0, 0), memory_space=pltpu.SMEM),
        pl.BlockSpec((1, 1), lambda i: (0, 0), memory_space=pltpu.SMEM),
    ],
    out_shape=[
        jax.ShapeDtypeStruct((1, 1), jnp.float32),
        jax.ShapeDtypeStruct((1, 1), jnp.float32),
    ],
)


def kernel(target_deltas, target_scores, output_deltas, output_scores):
    od = output_deltas.reshape(-1)
    td = target_deltas.reshape(-1)
    osf = output_scores.reshape(-1)
    ts2 = target_scores.reshape(ROWS, 128)
    os2 = output_scores.reshape(ROWS, 128)

    parts = _reg_call(od, td, osf)
    bce_sum, cnt_sum = _cls_call(ts2, os2)

    sums = jnp.sum(parts, axis=(0, 2))
    cls_loss = bce_sum[0, 0] / jnp.maximum(cnt_sum[0, 0], 1.0)
    reg_loss = 10.0 * sums[0] / (sums[1] + EPS * sums[2])
    return cls_loss + reg_loss


# speedup vs baseline: 38.2719x; 38.2719x over previous
import jax
import jax.numpy as jnp
from jax import lax
from jax.experimental import pallas as pl
from jax.experimental.pallas import tpu as pltpu
from jax.experimental.pallas import tpu_sc as plsc

EPS = 1e-7
N = 589824
NC, NS, L = 2, 16, 16
NW = NC * NS
NA = N // NW
NCHUNK = 4
CH = NA // NCHUNK
GROUPS = CH // L


def _reg_body(od_hbm, td_hbm, os_hbm, out_hbm,
              od_v0, od_v1, td_v0, td_v1, os_v0, os_v1, acc_v,
              sem0, sem1):
    wid = lax.axis_index("c") * NS + lax.axis_index("s")
    od_bufs = (od_v0, od_v1)
    td_bufs = (td_v0, td_v1)
    os_bufs = (os_v0, os_v1)
    sems = (sem0, sem1)

    def chunk_copies(g):
        slot = g % 2
        base = pl.multiple_of(wid * NA + g * CH, 8)
        base4 = pl.multiple_of(base * 4, 8)
        return (
            pltpu.make_async_copy(od_hbm.at[pl.ds(base4, 4 * CH)],
                                  od_bufs[slot], sems[slot]),
            pltpu.make_async_copy(td_hbm.at[pl.ds(base4, 4 * CH)],
                                  td_bufs[slot], sems[slot]),
            pltpu.make_async_copy(os_hbm.at[pl.ds(base, CH)],
                                  os_bufs[slot], sems[slot]),
        )

    for c in chunk_copies(0):
        c.start()

    acc_a = jnp.zeros((L,), jnp.float32)
    acc_p = jnp.zeros((L,), jnp.float32)
    acc_m = jnp.zeros((L,), jnp.float32)

    for g in range(NCHUNK):
        slot = g % 2
        if g + 1 < NCHUNK:
            for c in chunk_copies(g + 1):
                c.start()
        for c in chunk_copies(g):
            c.wait()
        odv, tdv, osv = od_bufs[slot], td_bufs[slot], os_bufs[slot]

        def group(g2, carry, odv=odv, tdv=tdv, osv=osv):
            a, p, m = carry
            osg = osv[pl.ds(g2 * L, L)]
            w = jnp.where(osg > 0.0, 1.0, 0.0)
            p = p + w
            m = m + jnp.where(osg != -1.0, 1.0, 0.0)
            off = 512 * (g2 >> 3) + 16 * (g2 & 7)
            for c in range(4):
                d = jnp.abs(odv[pl.ds(off + c * 128, L)]
                            - tdv[pl.ds(off + c * 128, L)])
                mn = jnp.minimum(d, 1.0)
                a = a + (0.5 * mn * mn + (d - mn)) * w
            return a, p, m

        acc_a, acc_p, acc_m = lax.fori_loop(
            0, GROUPS, group, (acc_a, acc_p, acc_m))

    acc_v[0, :] = acc_a
    acc_v[1, :] = acc_p
    acc_v[2, :] = acc_m
    pltpu.sync_copy(acc_v, out_hbm.at[wid])


_reg_call = pl.kernel(
    _reg_body,
    out_type=jax.ShapeDtypeStruct((NW, 3, L), jnp.float32),
    mesh=plsc.VectorSubcoreMesh(core_axis_name="c", subcore_axis_name="s"),
    compiler_params=pltpu.CompilerParams(needs_layout_passes=False),
    scratch_types=[
        pltpu.VMEM((4 * CH,), jnp.float32),
        pltpu.VMEM((4 * CH,), jnp.float32),
        pltpu.VMEM((4 * CH,), jnp.float32),
        pltpu.VMEM((4 * CH,), jnp.float32),
        pltpu.VMEM((CH,), jnp.float32),
        pltpu.VMEM((CH,), jnp.float32),
        pltpu.VMEM((3, L), jnp.float32),
        pltpu.SemaphoreType.DMA,
        pltpu.SemaphoreType.DMA,
    ],
)


ROWS = N // 128
TC_GRID = 16
TC_BLK = ROWS // TC_GRID


def _cls_body(ts_ref, os_ref, bce_ref, cnt_ref):
    i = pl.program_id(0)
    ts = ts_ref[...]
    o = jnp.clip(os_ref[...], EPS, 1.0 - EPS)
    mask = (ts != -1.0).astype(jnp.float32)
    bce = -(ts * jnp.log(o) + (1.0 - ts) * jnp.log(1.0 - o))

    @pl.when(i == 0)
    def _():
        bce_ref[0, 0] = 0.0
        cnt_ref[0, 0] = 0.0

    bce_ref[0, 0] += jnp.sum(bce * mask)
    cnt_ref[0, 0] += jnp.sum(mask)


_cls_call = pl.pallas_call(
    _cls_body,
    grid=(TC_GRID,),
    in_specs=[
        pl.BlockSpec((TC_BLK, 128), lambda i: (i, 0)),
        pl.BlockSpec((TC_BLK, 128), lambda i: (i, 0)),
    ],
    out_specs=[
        pl.BlockSpec((1, 1), lambda i: (0, 0), memory_space=pltpu.SMEM),
        pl.BlockSpec((1, 1), lambda i: (0, 0), memory_space=pltpu.SMEM),
    ],
    out_shape=[
        jax.ShapeDtypeStruct((1, 1), jnp.float32),
        jax.ShapeDtypeStruct((1, 1), jnp.float32),
    ],
)


def _planar_flat(x):
    return x.reshape(N // 128, 128, 4).transpose(0, 2, 1).reshape(-1)


def kernel(target_deltas, target_scores, output_deltas, output_scores):
    od = _planar_flat(output_deltas)
    td = _planar_flat(target_deltas)
    osf = output_scores.reshape(-1)
    ts2 = target_scores.reshape(ROWS, 128)
    os2 = output_scores.reshape(ROWS, 128)

    parts = _reg_call(od, td, osf)
    bce_sum, cnt_sum = _cls_call(ts2, os2)

    sums = jnp.sum(parts, axis=(0, 2))
    cls_loss = bce_sum[0, 0] / jnp.maximum(cnt_sum[0, 0], 1.0)
    reg_loss = 10.0 * sums[0] / (sums[1] + EPS * sums[2])
    return cls_loss + reg_loss
